# baseline (speedup 1.0000x reference)
import functools

import jax
import jax.numpy as jnp
from jax import lax
from jax.experimental import pallas as pl
from jax.experimental.pallas import tpu as pltpu
from jax.experimental.pallas import tpu_sc as plsc

NN = 10000
NNP = 10240
DH = 512
NCH = 4
CW = 128
EE = 160000
NC = 2
NS = 16
RPT = NNP // NS
EPT = EE // NS
EPD = EE // (NC * NS)
EB = 128
NB = 80
BLK = 8
NBLK = NB // BLK
NPAD = NB * EB - EPT
BN = 2000

_MESH = plsc.VectorSubcoreMesh(
    core_axis_name="c", subcore_axis_name="s", num_cores=NC, num_subcores=NS)


def _deg_body(dst_hbm, ones_hbm, zeros_hbm, out_hbm,
              ones_v, stage_v, idx_v, idxt_v, deg_sh):
    c = lax.axis_index("c")
    s = lax.axis_index("s")
    pltpu.sync_copy(ones_hbm, ones_v)
    pltpu.sync_copy(zeros_hbm, stage_v)
    for j in range(RPT // EB):
        pltpu.sync_copy(stage_v, deg_sh.at[pl.ds(s * RPT + j * EB, EB)])
    plsc.subcore_barrier()
    base = (c * NS + s) * EPD
    nfull = EPD // EB
    tail = EPD - nfull * EB

    def ebody(i, carry):
        pltpu.sync_copy(dst_hbm.at[pl.ds(base + i * EB, EB)], idx_v)
        pltpu.sync_copy(ones_v, deg_sh.at[idx_v], add=True)
        return carry

    lax.fori_loop(0, nfull, ebody, 0)
    pltpu.sync_copy(dst_hbm.at[pl.ds(base + nfull * EB, tail)], idxt_v)
    pltpu.sync_copy(ones_v.at[pl.ds(0, tail)], deg_sh.at[idxt_v], add=True)
    plsc.subcore_barrier()
    for j in range(RPT // EB):
        pltpu.sync_copy(deg_sh.at[pl.ds(s * RPT + j * EB, EB)], stage_v)
        pltpu.sync_copy(stage_v, out_hbm.at[pl.ds(c * NNP + s * RPT + j * EB, EB)])


_deg_call = pl.kernel(
    _deg_body,
    out_type=jax.ShapeDtypeStruct((NC * NNP, CW), jnp.float32),
    mesh=_MESH,
    scratch_types=[
        pltpu.VMEM((EB, CW), jnp.float32),
        pltpu.VMEM((EB, CW), jnp.float32),
        pltpu.VMEM((EB,), jnp.int32),
        pltpu.VMEM((8,), jnp.int32),
        pltpu.VMEM_SHARED((NNP, CW), jnp.float32),
    ],
)


def _prop_body(y_hbm, sd2_hbm, out_hbm,
               rows0, rows1, sdblk, acc_sh, semg0, semg1, sems0, sems1):
    c = lax.axis_index("c")
    s = lax.axis_index("s")
    bufs = (rows0, rows1)
    gsems = (semg0, semg1)
    ssems = (sems0, sems1)
    npc = RPT // EB
    for k in range(NCH // NC):
        chunk = c + NC * k
        ybase = chunk * NNP
        h = pltpu.async_copy(y_hbm.at[pl.ds(ybase + s * RPT, EB)], rows0, semg0)
        for j in range(1, npc + 1):
            if j < npc:
                h2 = pltpu.async_copy(
                    y_hbm.at[pl.ds(ybase + s * RPT + j * EB, EB)],
                    bufs[j % 2], gsems[j % 2])
            h.wait()
            pltpu.sync_copy(bufs[(j - 1) % 2],
                            acc_sh.at[pl.ds(s * RPT + (j - 1) * EB, EB)])
            if j < npc:
                h = h2
        plsc.subcore_barrier()
        sdbase = ((chunk * NS + s) * NBLK) * 2 * BLK

        def blk_body(b, carry):
            pltpu.sync_copy(sd2_hbm.at[pl.ds(sdbase + b * 2 * BLK, 2 * BLK)],
                            sdblk)
            hs = [None, None]
            for j in range(BLK):
                p = j % 2
                if hs[p] is not None:
                    hs[p].wait()
                pltpu.async_copy(y_hbm.at[sdblk.at[j]], bufs[p],
                                 gsems[p]).wait()
                hs[p] = pltpu.async_copy(bufs[p], acc_sh.at[sdblk.at[BLK + j]],
                                         ssems[p], add=True)
            hs[0].wait()
            hs[1].wait()
            return carry

        lax.fori_loop(0, NBLK, blk_body, 0)
        plsc.subcore_barrier()
        h = pltpu.async_copy(acc_sh.at[pl.ds(s * RPT, EB)], rows0, semg0)
        for j in range(1, npc + 1):
            if j < npc:
                h2 = pltpu.async_copy(acc_sh.at[pl.ds(s * RPT + j * EB, EB)],
                                      bufs[j % 2], gsems[j % 2])
            h.wait()
            pltpu.sync_copy(bufs[(j - 1) % 2],
                            out_hbm.at[pl.ds(ybase + s * RPT + (j - 1) * EB, EB)])
            if j < npc:
                h = h2
        if k + 1 < NCH // NC:
            plsc.subcore_barrier()


_prop_call = pl.kernel(
    _prop_body,
    out_type=jax.ShapeDtypeStruct((NCH * NNP, CW), jnp.float32),
    mesh=_MESH,
    scratch_types=[
        pltpu.VMEM((EB, CW), jnp.float32),
        pltpu.VMEM((EB, CW), jnp.float32),
        pltpu.VMEM((2 * BLK, EB), jnp.int32),
        pltpu.VMEM_SHARED((NNP, CW), jnp.float32),
        pltpu.SemaphoreType.DMA,
        pltpu.SemaphoreType.DMA,
        pltpu.SemaphoreType.DMA,
        pltpu.SemaphoreType.DMA,
    ],
)


def _a1_body(x_ref, w_ref, deg_ref, y_ref, dis_ref):
    degsum = deg_ref[0, :, 0:1] + deg_ref[1, :, 0:1] + 1.0
    dis = lax.rsqrt(degsum)
    dis_ref[...] = dis
    y = jnp.dot(x_ref[...], w_ref[...], preferred_element_type=jnp.float32) * dis
    for ch in range(NCH):
        y_ref[ch, :, :] = y[:, ch * CW:(ch + 1) * CW]


def _a1_call(x, w, deg2):
    return pl.pallas_call(
        _a1_body,
        grid=(NN // BN,),
        in_specs=[
            pl.BlockSpec((BN, x.shape[1]), lambda i: (i, 0)),
            pl.BlockSpec(w.shape, lambda i: (0, 0)),
            pl.BlockSpec((NC, BN, CW), lambda i: (0, i, 0)),
        ],
        out_specs=[
            pl.BlockSpec((NCH, BN, CW), lambda i: (0, i, 0)),
            pl.BlockSpec((BN, 1), lambda i: (i, 0)),
        ],
        out_shape=[
            jax.ShapeDtypeStruct((NCH, NNP, CW), jnp.float32),
            jax.ShapeDtypeStruct((NN, 1), jnp.float32),
        ],
    )(x, w, deg2)


def _amid_body(acc_ref, dis_ref, b_ref, w_ref, y_ref):
    dis = dis_ref[...]
    ysum = jnp.zeros((BN, DH), jnp.float32)
    for ch in range(NCH):
        h = jnp.maximum(acc_ref[ch] * dis + b_ref[0, ch * CW:(ch + 1) * CW], 0.0)
        ysum += jnp.dot(h, w_ref[ch * CW:(ch + 1) * CW, :],
                        preferred_element_type=jnp.float32)
    y = ysum * dis
    for ch in range(NCH):
        y_ref[ch, :, :] = y[:, ch * CW:(ch + 1) * CW]


def _amid_call(acc, dis, b, w):
    return pl.pallas_call(
        _amid_body,
        grid=(NN // BN,),
        in_specs=[
            pl.BlockSpec((NCH, BN, CW), lambda i: (0, i, 0)),
            pl.BlockSpec((BN, 1), lambda i: (i, 0)),
            pl.BlockSpec((1, DH), lambda i: (0, 0)),
            pl.BlockSpec((DH, DH), lambda i: (0, 0)),
        ],
        out_specs=pl.BlockSpec((NCH, BN, CW), lambda i: (0, i, 0)),
        out_shape=jax.ShapeDtypeStruct((NCH, NNP, CW), jnp.float32),
    )(acc, dis, b, w)


def _head_body(acc_ref, dis_ref, b3_ref, w1_ref, l1b_ref, w2_ref, l2b_ref, o_ref):
    dis = dis_ref[...]
    z = jnp.zeros((BN, DH), jnp.float32)
    for ch in range(NCH):
        h = jnp.maximum(acc_ref[ch] * dis + b3_ref[0, ch * CW:(ch + 1) * CW], 0.0)
        z += jnp.dot(h, w1_ref[ch * CW:(ch + 1) * CW, :],
                     preferred_element_type=jnp.float32)
    z = jnp.maximum(z + l1b_ref[...], 0.0)
    o_ref[...] = jnp.dot(z, w2_ref[...], preferred_element_type=jnp.float32) \
        + l2b_ref[...]


def _head_call(acc, dis, b3, w1, l1b, w2, l2b):
    ncls = w2.shape[1]
    return pl.pallas_call(
        _head_body,
        grid=(NN // BN,),
        in_specs=[
            pl.BlockSpec((NCH, BN, CW), lambda i: (0, i, 0)),
            pl.BlockSpec((BN, 1), lambda i: (i, 0)),
            pl.BlockSpec((1, DH), lambda i: (0, 0)),
            pl.BlockSpec((DH, DH), lambda i: (0, 0)),
            pl.BlockSpec((1, DH), lambda i: (0, 0)),
            pl.BlockSpec((DH, ncls), lambda i: (0, 0)),
            pl.BlockSpec((1, ncls), lambda i: (0, 0)),
        ],
        out_specs=pl.BlockSpec((BN, ncls), lambda i: (i, 0)),
        out_shape=jax.ShapeDtypeStruct((NN, ncls), jnp.float32),
    )(acc, dis, b3, w1, l1b, w2, l2b)


def kernel(dataX, dataY, W1, b1, W2, b2, W3, b3, lin1_W, lin1_b, lin2_W, lin2_b):
    src = dataY[0].astype(jnp.int32)
    dst = dataY[1].astype(jnp.int32)
    pad_rows = jnp.arange(NPAD, dtype=jnp.int32) + NN
    padb = jnp.broadcast_to(pad_rows, (NS, NPAD))
    srcp = jnp.concatenate([src.reshape(NS, EPT), padb], axis=1)
    dstp = jnp.concatenate([dst.reshape(NS, EPT), padb], axis=1)
    srcb = srcp.reshape(NS, NBLK, BLK, EB)
    dstb = dstp.reshape(NS, NBLK, BLK, EB)
    src2c = (srcb[None]
             + (jnp.arange(NCH, dtype=jnp.int32) * NNP)[:, None, None, None, None])
    dst2c = jnp.broadcast_to(dstb[None], (NCH, NS, NBLK, BLK, EB))
    sd2 = jnp.concatenate([src2c, dst2c], axis=3).reshape(-1, EB)
    ones_rows = jnp.ones((EB, CW), jnp.float32)
    zeros_rows = jnp.zeros((EB, CW), jnp.float32)

    deg2 = _deg_call(dst, ones_rows, zeros_rows).reshape(NC, NNP, CW)
    y1, dis = _a1_call(dataX, W1, deg2)
    acc1 = _prop_call(y1.reshape(NCH * NNP, CW), sd2)
    y2 = _amid_call(acc1.reshape(NCH, NNP, CW), dis, b1.reshape(1, DH), W2)
    acc2 = _prop_call(y2.reshape(NCH * NNP, CW), sd2)
    y3 = _amid_call(acc2.reshape(NCH, NNP, CW), dis, b2.reshape(1, DH), W3)
    acc3 = _prop_call(y3.reshape(NCH * NNP, CW), sd2)
    return _head_call(acc3.reshape(NCH, NNP, CW), dis, b3.reshape(1, DH),
                      lin1_W, lin1_b.reshape(1, DH),
                      lin2_W, lin2_b.reshape(1, lin2_W.shape[1]))


# speedup vs baseline: 1.0930x; 1.0930x over previous
import functools

import jax
import jax.numpy as jnp
from jax import lax
from jax.experimental import pallas as pl
from jax.experimental.pallas import tpu as pltpu
from jax.experimental.pallas import tpu_sc as plsc

NN = 10000
NNP = 10240
DH = 512
NCH = 4
CW = 128
EE = 160000
NC = 2
NS = 16
RPT = NNP // NS
EPT = EE // NS
EPD = EE // (NC * NS)
EB = 128
NB = 80
BLK = 8
NBLK = NB // BLK
NPAD = NB * EB - EPT
BN = 2000

_MESH = plsc.VectorSubcoreMesh(
    core_axis_name="c", subcore_axis_name="s", num_cores=NC, num_subcores=NS)


def _deg_body(dst_hbm, ones_hbm, zeros_hbm, out_hbm,
              ones_v, stage_v, idx_v, idxt_v, deg_sh):
    c = lax.axis_index("c")
    s = lax.axis_index("s")
    pltpu.sync_copy(ones_hbm, ones_v)
    pltpu.sync_copy(zeros_hbm, stage_v)
    for j in range(RPT // EB):
        pltpu.sync_copy(stage_v, deg_sh.at[pl.ds(s * RPT + j * EB, EB)])
    plsc.subcore_barrier()
    base = (c * NS + s) * EPD
    nfull = EPD // EB
    tail = EPD - nfull * EB

    def ebody(i, carry):
        pltpu.sync_copy(dst_hbm.at[pl.ds(base + i * EB, EB)], idx_v)
        pltpu.sync_copy(ones_v, deg_sh.at[idx_v], add=True)
        return carry

    lax.fori_loop(0, nfull, ebody, 0)
    pltpu.sync_copy(dst_hbm.at[pl.ds(base + nfull * EB, tail)], idxt_v)
    pltpu.sync_copy(ones_v.at[pl.ds(0, tail)], deg_sh.at[idxt_v], add=True)
    plsc.subcore_barrier()
    for j in range(RPT // EB):
        pltpu.sync_copy(deg_sh.at[pl.ds(s * RPT + j * EB, EB)], stage_v)
        pltpu.sync_copy(stage_v, out_hbm.at[pl.ds(c * NNP + s * RPT + j * EB, EB)])


_deg_call = pl.kernel(
    _deg_body,
    out_type=jax.ShapeDtypeStruct((NC * NNP, CW), jnp.float32),
    mesh=_MESH,
    scratch_types=[
        pltpu.VMEM((EB, CW), jnp.float32),
        pltpu.VMEM((EB, CW), jnp.float32),
        pltpu.VMEM((EB,), jnp.int32),
        pltpu.VMEM((8,), jnp.int32),
        pltpu.VMEM_SHARED((NNP, CW), jnp.float32),
    ],
)


def _prop_body(y_hbm, sd2_hbm, out_hbm,
               rows0, rows1, sdblk, acc_sh, semg0, semg1, sems0, sems1):
    c = lax.axis_index("c")
    s = lax.axis_index("s")
    bufs = (rows0, rows1)
    gsems = (semg0, semg1)
    ssems = (sems0, sems1)
    npc = RPT // EB
    for k in range(NCH // NC):
        chunk = c + NC * k
        ybase = chunk * NNP
        h = pltpu.async_copy(y_hbm.at[pl.ds(ybase + s * RPT, EB)], rows0, semg0)
        for j in range(1, npc + 1):
            if j < npc:
                h2 = pltpu.async_copy(
                    y_hbm.at[pl.ds(ybase + s * RPT + j * EB, EB)],
                    bufs[j % 2], gsems[j % 2])
            h.wait()
            pltpu.sync_copy(bufs[(j - 1) % 2],
                            acc_sh.at[pl.ds(s * RPT + (j - 1) * EB, EB)])
            if j < npc:
                h = h2
        plsc.subcore_barrier()
        sdbase = ((chunk * NS + s) * NBLK) * 2 * BLK

        def blk_body(b, carry):
            pltpu.sync_copy(sd2_hbm.at[pl.ds(sdbase + b * 2 * BLK, 2 * BLK)],
                            sdblk)
            hs = [None, None]
            g = pltpu.async_copy(y_hbm.at[sdblk.at[0]], bufs[0], gsems[0])
            for j in range(BLK):
                p = j % 2
                q = (j + 1) % 2
                if j + 1 < BLK:
                    if hs[q] is not None:
                        hs[q].wait()
                    g2 = pltpu.async_copy(y_hbm.at[sdblk.at[j + 1]],
                                          bufs[q], gsems[q])
                g.wait()
                hs[p] = pltpu.async_copy(bufs[p], acc_sh.at[sdblk.at[BLK + j]],
                                         ssems[p], add=True)
                if j + 1 < BLK:
                    g = g2
            hs[0].wait()
            hs[1].wait()
            return carry

        lax.fori_loop(0, NBLK, blk_body, 0)
        plsc.subcore_barrier()
        h = pltpu.async_copy(acc_sh.at[pl.ds(s * RPT, EB)], rows0, semg0)
        for j in range(1, npc + 1):
            if j < npc:
                h2 = pltpu.async_copy(acc_sh.at[pl.ds(s * RPT + j * EB, EB)],
                                      bufs[j % 2], gsems[j % 2])
            h.wait()
            pltpu.sync_copy(bufs[(j - 1) % 2],
                            out_hbm.at[pl.ds(ybase + s * RPT + (j - 1) * EB, EB)])
            if j < npc:
                h = h2
        if k + 1 < NCH // NC:
            plsc.subcore_barrier()


_prop_call = pl.kernel(
    _prop_body,
    out_type=jax.ShapeDtypeStruct((NCH * NNP, CW), jnp.float32),
    mesh=_MESH,
    scratch_types=[
        pltpu.VMEM((EB, CW), jnp.float32),
        pltpu.VMEM((EB, CW), jnp.float32),
        pltpu.VMEM((2 * BLK, EB), jnp.int32),
        pltpu.VMEM_SHARED((NNP, CW), jnp.float32),
        pltpu.SemaphoreType.DMA,
        pltpu.SemaphoreType.DMA,
        pltpu.SemaphoreType.DMA,
        pltpu.SemaphoreType.DMA,
    ],
)


def _a1_body(x_ref, w_ref, deg_ref, y_ref, dis_ref):
    degsum = deg_ref[0, :, 0:1] + deg_ref[1, :, 0:1] + 1.0
    dis = lax.rsqrt(degsum)
    dis_ref[...] = dis
    y = jnp.dot(x_ref[...], w_ref[...], preferred_element_type=jnp.float32) * dis
    for ch in range(NCH):
        y_ref[ch, :, :] = y[:, ch * CW:(ch + 1) * CW]


def _a1_call(x, w, deg2):
    return pl.pallas_call(
        _a1_body,
        grid=(NN // BN,),
        in_specs=[
            pl.BlockSpec((BN, x.shape[1]), lambda i: (i, 0)),
            pl.BlockSpec(w.shape, lambda i: (0, 0)),
            pl.BlockSpec((NC, BN, CW), lambda i: (0, i, 0)),
        ],
        out_specs=[
            pl.BlockSpec((NCH, BN, CW), lambda i: (0, i, 0)),
            pl.BlockSpec((BN, 1), lambda i: (i, 0)),
        ],
        out_shape=[
            jax.ShapeDtypeStruct((NCH, NNP, CW), jnp.float32),
            jax.ShapeDtypeStruct((NN, 1), jnp.float32),
        ],
    )(x, w, deg2)


def _amid_body(acc_ref, dis_ref, b_ref, w_ref, y_ref):
    dis = dis_ref[...]
    ysum = jnp.zeros((BN, DH), jnp.float32)
    for ch in range(NCH):
        h = jnp.maximum(acc_ref[ch] * dis + b_ref[0, ch * CW:(ch + 1) * CW], 0.0)
        ysum += jnp.dot(h, w_ref[ch * CW:(ch + 1) * CW, :],
                        preferred_element_type=jnp.float32)
    y = ysum * dis
    for ch in range(NCH):
        y_ref[ch, :, :] = y[:, ch * CW:(ch + 1) * CW]


def _amid_call(acc, dis, b, w):
    return pl.pallas_call(
        _amid_body,
        grid=(NN // BN,),
        in_specs=[
            pl.BlockSpec((NCH, BN, CW), lambda i: (0, i, 0)),
            pl.BlockSpec((BN, 1), lambda i: (i, 0)),
            pl.BlockSpec((1, DH), lambda i: (0, 0)),
            pl.BlockSpec((DH, DH), lambda i: (0, 0)),
        ],
        out_specs=pl.BlockSpec((NCH, BN, CW), lambda i: (0, i, 0)),
        out_shape=jax.ShapeDtypeStruct((NCH, NNP, CW), jnp.float32),
    )(acc, dis, b, w)


def _head_body(acc_ref, dis_ref, b3_ref, w1_ref, l1b_ref, w2_ref, l2b_ref, o_ref):
    dis = dis_ref[...]
    z = jnp.zeros((BN, DH), jnp.float32)
    for ch in range(NCH):
        h = jnp.maximum(acc_ref[ch] * dis + b3_ref[0, ch * CW:(ch + 1) * CW], 0.0)
        z += jnp.dot(h, w1_ref[ch * CW:(ch + 1) * CW, :],
                     preferred_element_type=jnp.float32)
    z = jnp.maximum(z + l1b_ref[...], 0.0)
    o_ref[...] = jnp.dot(z, w2_ref[...], preferred_element_type=jnp.float32) \
        + l2b_ref[...]


def _head_call(acc, dis, b3, w1, l1b, w2, l2b):
    ncls = w2.shape[1]
    return pl.pallas_call(
        _head_body,
        grid=(NN // BN,),
        in_specs=[
            pl.BlockSpec((NCH, BN, CW), lambda i: (0, i, 0)),
            pl.BlockSpec((BN, 1), lambda i: (i, 0)),
            pl.BlockSpec((1, DH), lambda i: (0, 0)),
            pl.BlockSpec((DH, DH), lambda i: (0, 0)),
            pl.BlockSpec((1, DH), lambda i: (0, 0)),
            pl.BlockSpec((DH, ncls), lambda i: (0, 0)),
            pl.BlockSpec((1, ncls), lambda i: (0, 0)),
        ],
        out_specs=pl.BlockSpec((BN, ncls), lambda i: (i, 0)),
        out_shape=jax.ShapeDtypeStruct((NN, ncls), jnp.float32),
    )(acc, dis, b3, w1, l1b, w2, l2b)


def kernel(dataX, dataY, W1, b1, W2, b2, W3, b3, lin1_W, lin1_b, lin2_W, lin2_b):
    src = dataY[0].astype(jnp.int32)
    dst = dataY[1].astype(jnp.int32)
    pad_rows = jnp.arange(NPAD, dtype=jnp.int32) + NN
    padb = jnp.broadcast_to(pad_rows, (NS, NPAD))
    srcp = jnp.concatenate([src.reshape(NS, EPT), padb], axis=1)
    dstp = jnp.concatenate([dst.reshape(NS, EPT), padb], axis=1)
    srcb = srcp.reshape(NS, NBLK, BLK, EB)
    dstb = dstp.reshape(NS, NBLK, BLK, EB)
    src2c = (srcb[None]
             + (jnp.arange(NCH, dtype=jnp.int32) * NNP)[:, None, None, None, None])
    dst2c = jnp.broadcast_to(dstb[None], (NCH, NS, NBLK, BLK, EB))
    sd2 = jnp.concatenate([src2c, dst2c], axis=3).reshape(-1, EB)
    ones_rows = jnp.ones((EB, CW), jnp.float32)
    zeros_rows = jnp.zeros((EB, CW), jnp.float32)

    deg2 = _deg_call(dst, ones_rows, zeros_rows).reshape(NC, NNP, CW)
    y1, dis = _a1_call(dataX, W1, deg2)
    acc1 = _prop_call(y1.reshape(NCH * NNP, CW), sd2)
    y2 = _amid_call(acc1.reshape(NCH, NNP, CW), dis, b1.reshape(1, DH), W2)
    acc2 = _prop_call(y2.reshape(NCH * NNP, CW), sd2)
    y3 = _amid_call(acc2.reshape(NCH, NNP, CW), dis, b2.reshape(1, DH), W3)
    acc3 = _prop_call(y3.reshape(NCH * NNP, CW), sd2)
    return _head_call(acc3.reshape(NCH, NNP, CW), dis, b3.reshape(1, DH),
                      lin1_W, lin1_b.reshape(1, DH),
                      lin2_W, lin2_b.reshape(1, lin2_W.shape[1]))


# speedup vs baseline: 1.1698x; 1.0703x over previous
import functools

import jax
import jax.numpy as jnp
from jax import lax
from jax.experimental import pallas as pl
from jax.experimental.pallas import tpu as pltpu
from jax.experimental.pallas import tpu_sc as plsc

NN = 10000
NNP = 10240
DH = 512
NCH = 4
CW = 128
EE = 160000
NC = 2
NS = 16
RPT = NNP // NS
EPT = EE // NS
EPD = EE // (NC * NS)
EB = 128
NB = 80
BLK = 16
NBLK = NB // BLK
NPAD = NB * EB - EPT
BN = 2000

_MESH = plsc.VectorSubcoreMesh(
    core_axis_name="c", subcore_axis_name="s", num_cores=NC, num_subcores=NS)


def _deg_body(dst_hbm, ones_hbm, zeros_hbm, out_hbm,
              ones_v, stage_v, idx_v, idxt_v, deg_sh):
    c = lax.axis_index("c")
    s = lax.axis_index("s")
    pltpu.sync_copy(ones_hbm, ones_v)
    pltpu.sync_copy(zeros_hbm, stage_v)
    for j in range(RPT // EB):
        pltpu.sync_copy(stage_v, deg_sh.at[pl.ds(s * RPT + j * EB, EB)])
    plsc.subcore_barrier()
    base = (c * NS + s) * EPD
    nfull = EPD // EB
    tail = EPD - nfull * EB

    def ebody(i, carry):
        pltpu.sync_copy(dst_hbm.at[pl.ds(base + i * EB, EB)], idx_v)
        pltpu.sync_copy(ones_v, deg_sh.at[idx_v], add=True)
        return carry

    lax.fori_loop(0, nfull, ebody, 0)
    pltpu.sync_copy(dst_hbm.at[pl.ds(base + nfull * EB, tail)], idxt_v)
    pltpu.sync_copy(ones_v.at[pl.ds(0, tail)], deg_sh.at[idxt_v], add=True)
    plsc.subcore_barrier()
    for j in range(RPT // EB):
        pltpu.sync_copy(deg_sh.at[pl.ds(s * RPT + j * EB, EB)], stage_v)
        pltpu.sync_copy(stage_v, out_hbm.at[pl.ds(c * NNP + s * RPT + j * EB, EB)])


_deg_call = pl.kernel(
    _deg_body,
    out_type=jax.ShapeDtypeStruct((NC * NNP, CW), jnp.float32),
    mesh=_MESH,
    scratch_types=[
        pltpu.VMEM((EB, CW), jnp.float32),
        pltpu.VMEM((EB, CW), jnp.float32),
        pltpu.VMEM((EB,), jnp.int32),
        pltpu.VMEM((8,), jnp.int32),
        pltpu.VMEM_SHARED((NNP, CW), jnp.float32),
    ],
)


def _prop_body(y_hbm, sd2_hbm, out_hbm,
               rows0, rows1, sdblk, acc_sh, semg0, semg1, sems0, sems1):
    c = lax.axis_index("c")
    s = lax.axis_index("s")
    bufs = (rows0, rows1)
    gsems = (semg0, semg1)
    ssems = (sems0, sems1)
    npc = RPT // EB
    for k in range(NCH // NC):
        chunk = c + NC * k
        ybase = chunk * NNP
        h = pltpu.async_copy(y_hbm.at[pl.ds(ybase + s * RPT, EB)], rows0, semg0)
        for j in range(1, npc + 1):
            if j < npc:
                h2 = pltpu.async_copy(
                    y_hbm.at[pl.ds(ybase + s * RPT + j * EB, EB)],
                    bufs[j % 2], gsems[j % 2])
            h.wait()
            pltpu.sync_copy(bufs[(j - 1) % 2],
                            acc_sh.at[pl.ds(s * RPT + (j - 1) * EB, EB)])
            if j < npc:
                h = h2
        plsc.subcore_barrier()
        sdbase = ((chunk * NS + s) * NBLK) * 2 * BLK

        def blk_body(b, carry):
            pltpu.sync_copy(sd2_hbm.at[pl.ds(sdbase + b * 2 * BLK, 2 * BLK)],
                            sdblk)
            hs = [None, None]
            g = pltpu.async_copy(y_hbm.at[sdblk.at[0]], bufs[0], gsems[0])
            for j in range(BLK):
                p = j % 2
                q = (j + 1) % 2
                if j + 1 < BLK:
                    if hs[q] is not None:
                        hs[q].wait()
                    g2 = pltpu.async_copy(y_hbm.at[sdblk.at[j + 1]],
                                          bufs[q], gsems[q])
                g.wait()
                hs[p] = pltpu.async_copy(bufs[p], acc_sh.at[sdblk.at[BLK + j]],
                                         ssems[p], add=True)
                if j + 1 < BLK:
                    g = g2
            hs[0].wait()
            hs[1].wait()
            return carry

        lax.fori_loop(0, NBLK, blk_body, 0)
        plsc.subcore_barrier()
        h = pltpu.async_copy(acc_sh.at[pl.ds(s * RPT, EB)], rows0, semg0)
        for j in range(1, npc + 1):
            if j < npc:
                h2 = pltpu.async_copy(acc_sh.at[pl.ds(s * RPT + j * EB, EB)],
                                      bufs[j % 2], gsems[j % 2])
            h.wait()
            pltpu.sync_copy(bufs[(j - 1) % 2],
                            out_hbm.at[pl.ds(ybase + s * RPT + (j - 1) * EB, EB)])
            if j < npc:
                h = h2
        if k + 1 < NCH // NC:
            plsc.subcore_barrier()


_prop_call = pl.kernel(
    _prop_body,
    out_type=jax.ShapeDtypeStruct((NCH * NNP, CW), jnp.float32),
    mesh=_MESH,
    scratch_types=[
        pltpu.VMEM((EB, CW), jnp.float32),
        pltpu.VMEM((EB, CW), jnp.float32),
        pltpu.VMEM((2 * BLK, EB), jnp.int32),
        pltpu.VMEM_SHARED((NNP, CW), jnp.float32),
        pltpu.SemaphoreType.DMA,
        pltpu.SemaphoreType.DMA,
        pltpu.SemaphoreType.DMA,
        pltpu.SemaphoreType.DMA,
    ],
)


def _a1_body(x_ref, w_ref, deg_ref, y_ref, dis_ref):
    degsum = deg_ref[0, :, 0:1] + deg_ref[1, :, 0:1] + 1.0
    dis = lax.rsqrt(degsum)
    dis_ref[...] = dis
    y = jnp.dot(x_ref[...], w_ref[...], preferred_element_type=jnp.float32) * dis
    for ch in range(NCH):
        y_ref[ch, :, :] = y[:, ch * CW:(ch + 1) * CW]


def _a1_call(x, w, deg2):
    return pl.pallas_call(
        _a1_body,
        grid=(NN // BN,),
        in_specs=[
            pl.BlockSpec((BN, x.shape[1]), lambda i: (i, 0)),
            pl.BlockSpec(w.shape, lambda i: (0, 0)),
            pl.BlockSpec((NC, BN, CW), lambda i: (0, i, 0)),
        ],
        out_specs=[
            pl.BlockSpec((NCH, BN, CW), lambda i: (0, i, 0)),
            pl.BlockSpec((BN, 1), lambda i: (i, 0)),
        ],
        out_shape=[
            jax.ShapeDtypeStruct((NCH, NNP, CW), jnp.float32),
            jax.ShapeDtypeStruct((NN, 1), jnp.float32),
        ],
    )(x, w, deg2)


def _amid_body(acc_ref, dis_ref, b_ref, w_ref, y_ref):
    dis = dis_ref[...]
    ysum = jnp.zeros((BN, DH), jnp.float32)
    for ch in range(NCH):
        h = jnp.maximum(acc_ref[ch] * dis + b_ref[0, ch * CW:(ch + 1) * CW], 0.0)
        ysum += jnp.dot(h, w_ref[ch * CW:(ch + 1) * CW, :],
                        preferred_element_type=jnp.float32)
    y = ysum * dis
    for ch in range(NCH):
        y_ref[ch, :, :] = y[:, ch * CW:(ch + 1) * CW]


def _amid_call(acc, dis, b, w):
    return pl.pallas_call(
        _amid_body,
        grid=(NN // BN,),
        in_specs=[
            pl.BlockSpec((NCH, BN, CW), lambda i: (0, i, 0)),
            pl.BlockSpec((BN, 1), lambda i: (i, 0)),
            pl.BlockSpec((1, DH), lambda i: (0, 0)),
            pl.BlockSpec((DH, DH), lambda i: (0, 0)),
        ],
        out_specs=pl.BlockSpec((NCH, BN, CW), lambda i: (0, i, 0)),
        out_shape=jax.ShapeDtypeStruct((NCH, NNP, CW), jnp.float32),
    )(acc, dis, b, w)


def _head_body(acc_ref, dis_ref, b3_ref, w1_ref, l1b_ref, w2_ref, l2b_ref, o_ref):
    dis = dis_ref[...]
    z = jnp.zeros((BN, DH), jnp.float32)
    for ch in range(NCH):
        h = jnp.maximum(acc_ref[ch] * dis + b3_ref[0, ch * CW:(ch + 1) * CW], 0.0)
        z += jnp.dot(h, w1_ref[ch * CW:(ch + 1) * CW, :],
                     preferred_element_type=jnp.float32)
    z = jnp.maximum(z + l1b_ref[...], 0.0)
    o_ref[...] = jnp.dot(z, w2_ref[...], preferred_element_type=jnp.float32) \
        + l2b_ref[...]


def _head_call(acc, dis, b3, w1, l1b, w2, l2b):
    ncls = w2.shape[1]
    return pl.pallas_call(
        _head_body,
        grid=(NN // BN,),
        in_specs=[
            pl.BlockSpec((NCH, BN, CW), lambda i: (0, i, 0)),
            pl.BlockSpec((BN, 1), lambda i: (i, 0)),
            pl.BlockSpec((1, DH), lambda i: (0, 0)),
            pl.BlockSpec((DH, DH), lambda i: (0, 0)),
            pl.BlockSpec((1, DH), lambda i: (0, 0)),
            pl.BlockSpec((DH, ncls), lambda i: (0, 0)),
            pl.BlockSpec((1, ncls), lambda i: (0, 0)),
        ],
        out_specs=pl.BlockSpec((BN, ncls), lambda i: (i, 0)),
        out_shape=jax.ShapeDtypeStruct((NN, ncls), jnp.float32),
    )(acc, dis, b3, w1, l1b, w2, l2b)


def kernel(dataX, dataY, W1, b1, W2, b2, W3, b3, lin1_W, lin1_b, lin2_W, lin2_b):
    src = dataY[0].astype(jnp.int32)
    dst = dataY[1].astype(jnp.int32)
    pad_rows = jnp.arange(NPAD, dtype=jnp.int32) + NN
    padb = jnp.broadcast_to(pad_rows, (NS, NPAD))
    srcp = jnp.concatenate([src.reshape(NS, EPT), padb], axis=1)
    dstp = jnp.concatenate([dst.reshape(NS, EPT), padb], axis=1)
    srcb = srcp.reshape(NS, NBLK, BLK, EB)
    dstb = dstp.reshape(NS, NBLK, BLK, EB)
    src2c = (srcb[None]
             + (jnp.arange(NCH, dtype=jnp.int32) * NNP)[:, None, None, None, None])
    dst2c = jnp.broadcast_to(dstb[None], (NCH, NS, NBLK, BLK, EB))
    sd2 = jnp.concatenate([src2c, dst2c], axis=3).reshape(-1, EB)
    ones_rows = jnp.ones((EB, CW), jnp.float32)
    zeros_rows = jnp.zeros((EB, CW), jnp.float32)

    deg2 = _deg_call(dst, ones_rows, zeros_rows).reshape(NC, NNP, CW)
    y1, dis = _a1_call(dataX, W1, deg2)
    acc1 = _prop_call(y1.reshape(NCH * NNP, CW), sd2)
    y2 = _amid_call(acc1.reshape(NCH, NNP, CW), dis, b1.reshape(1, DH), W2)
    acc2 = _prop_call(y2.reshape(NCH * NNP, CW), sd2)
    y3 = _amid_call(acc2.reshape(NCH, NNP, CW), dis, b2.reshape(1, DH), W3)
    acc3 = _prop_call(y3.reshape(NCH * NNP, CW), sd2)
    return _head_call(acc3.reshape(NCH, NNP, CW), dis, b3.reshape(1, DH),
                      lin1_W, lin1_b.reshape(1, DH),
                      lin2_W, lin2_b.reshape(1, lin2_W.shape[1]))


# speedup vs baseline: 1.1854x; 1.0133x over previous
import functools

import jax
import jax.numpy as jnp
from jax import lax
from jax.experimental import pallas as pl
from jax.experimental.pallas import tpu as pltpu
from jax.experimental.pallas import tpu_sc as plsc

NN = 10000
NNP = 10240
DH = 512
NCH = 4
CW = 128
EE = 160000
NC = 2
NS = 16
RPT = NNP // NS
EPT = EE // NS
EPD = EE // (NC * NS)
EB = 128
NB = 80
BLK = 20
NBLK = NB // BLK
NPAD = NB * EB - EPT
BN = 2000

_MESH = plsc.VectorSubcoreMesh(
    core_axis_name="c", subcore_axis_name="s", num_cores=NC, num_subcores=NS)


def _deg_body(dst_hbm, ones_hbm, zeros_hbm, out_hbm,
              ones_v, stage_v, idx_v, idxt_v, deg_sh):
    c = lax.axis_index("c")
    s = lax.axis_index("s")
    pltpu.sync_copy(ones_hbm, ones_v)
    pltpu.sync_copy(zeros_hbm, stage_v)
    for j in range(RPT // EB):
        pltpu.sync_copy(stage_v, deg_sh.at[pl.ds(s * RPT + j * EB, EB)])
    plsc.subcore_barrier()
    base = (c * NS + s) * EPD
    nfull = EPD // EB
    tail = EPD - nfull * EB

    def ebody(i, carry):
        pltpu.sync_copy(dst_hbm.at[pl.ds(base + i * EB, EB)], idx_v)
        pltpu.sync_copy(ones_v, deg_sh.at[idx_v], add=True)
        return carry

    lax.fori_loop(0, nfull, ebody, 0)
    pltpu.sync_copy(dst_hbm.at[pl.ds(base + nfull * EB, tail)], idxt_v)
    pltpu.sync_copy(ones_v.at[pl.ds(0, tail)], deg_sh.at[idxt_v], add=True)
    plsc.subcore_barrier()
    for j in range(RPT // EB):
        pltpu.sync_copy(deg_sh.at[pl.ds(s * RPT + j * EB, EB)], stage_v)
        pltpu.sync_copy(stage_v, out_hbm.at[pl.ds(c * NNP + s * RPT + j * EB, EB)])


_deg_call = pl.kernel(
    _deg_body,
    out_type=jax.ShapeDtypeStruct((NC * NNP, CW), jnp.float32),
    mesh=_MESH,
    scratch_types=[
        pltpu.VMEM((EB, CW), jnp.float32),
        pltpu.VMEM((EB, CW), jnp.float32),
        pltpu.VMEM((EB,), jnp.int32),
        pltpu.VMEM((8,), jnp.int32),
        pltpu.VMEM_SHARED((NNP, CW), jnp.float32),
    ],
)


def _prop_body(y_hbm, sd2_hbm, out_hbm,
               rows0, rows1, sdblk, acc_sh, semg0, semg1, sems0, sems1):
    c = lax.axis_index("c")
    s = lax.axis_index("s")
    bufs = (rows0, rows1)
    gsems = (semg0, semg1)
    ssems = (sems0, sems1)
    npc = RPT // EB
    for k in range(NCH // NC):
        chunk = c + NC * k
        ybase = chunk * NNP
        h = pltpu.async_copy(y_hbm.at[pl.ds(ybase + s * RPT, EB)], rows0, semg0)
        for j in range(1, npc + 1):
            if j < npc:
                h2 = pltpu.async_copy(
                    y_hbm.at[pl.ds(ybase + s * RPT + j * EB, EB)],
                    bufs[j % 2], gsems[j % 2])
            h.wait()
            pltpu.sync_copy(bufs[(j - 1) % 2],
                            acc_sh.at[pl.ds(s * RPT + (j - 1) * EB, EB)])
            if j < npc:
                h = h2
        plsc.subcore_barrier()
        sdbase = ((chunk * NS + s) * NBLK) * 2 * BLK

        def blk_body(b, carry):
            pltpu.sync_copy(sd2_hbm.at[pl.ds(sdbase + b * 2 * BLK, 2 * BLK)],
                            sdblk)
            hs = [None, None]
            g = pltpu.async_copy(y_hbm.at[sdblk.at[0]], bufs[0], gsems[0])
            for j in range(BLK):
                p = j % 2
                q = (j + 1) % 2
                if j + 1 < BLK:
                    if hs[q] is not None:
                        hs[q].wait()
                    g2 = pltpu.async_copy(y_hbm.at[sdblk.at[j + 1]],
                                          bufs[q], gsems[q])
                g.wait()
                hs[p] = pltpu.async_copy(bufs[p], acc_sh.at[sdblk.at[BLK + j]],
                                         ssems[p], add=True)
                if j + 1 < BLK:
                    g = g2
            hs[0].wait()
            hs[1].wait()
            return carry

        lax.fori_loop(0, NBLK, blk_body, 0)
        plsc.subcore_barrier()
        h = pltpu.async_copy(acc_sh.at[pl.ds(s * RPT, EB)], rows0, semg0)
        for j in range(1, npc + 1):
            if j < npc:
                h2 = pltpu.async_copy(acc_sh.at[pl.ds(s * RPT + j * EB, EB)],
                                      bufs[j % 2], gsems[j % 2])
            h.wait()
            pltpu.sync_copy(bufs[(j - 1) % 2],
                            out_hbm.at[pl.ds(ybase + s * RPT + (j - 1) * EB, EB)])
            if j < npc:
                h = h2
        if k + 1 < NCH // NC:
            plsc.subcore_barrier()


_prop_call = pl.kernel(
    _prop_body,
    out_type=jax.ShapeDtypeStruct((NCH * NNP, CW), jnp.float32),
    mesh=_MESH,
    scratch_types=[
        pltpu.VMEM((EB, CW), jnp.float32),
        pltpu.VMEM((EB, CW), jnp.float32),
        pltpu.VMEM((2 * BLK, EB), jnp.int32),
        pltpu.VMEM_SHARED((NNP, CW), jnp.float32),
        pltpu.SemaphoreType.DMA,
        pltpu.SemaphoreType.DMA,
        pltpu.SemaphoreType.DMA,
        pltpu.SemaphoreType.DMA,
    ],
)


def _a1_body(x_ref, w_ref, deg_ref, y_ref, dis_ref):
    degsum = deg_ref[0, :, 0:1] + deg_ref[1, :, 0:1] + 1.0
    dis = lax.rsqrt(degsum)
    dis_ref[...] = dis
    y = jnp.dot(x_ref[...], w_ref[...], preferred_element_type=jnp.float32) * dis
    for ch in range(NCH):
        y_ref[ch, :, :] = y[:, ch * CW:(ch + 1) * CW]


def _a1_call(x, w, deg2):
    return pl.pallas_call(
        _a1_body,
        grid=(NN // BN,),
        in_specs=[
            pl.BlockSpec((BN, x.shape[1]), lambda i: (i, 0)),
            pl.BlockSpec(w.shape, lambda i: (0, 0)),
            pl.BlockSpec((NC, BN, CW), lambda i: (0, i, 0)),
        ],
        out_specs=[
            pl.BlockSpec((NCH, BN, CW), lambda i: (0, i, 0)),
            pl.BlockSpec((BN, 1), lambda i: (i, 0)),
        ],
        out_shape=[
            jax.ShapeDtypeStruct((NCH, NNP, CW), jnp.float32),
            jax.ShapeDtypeStruct((NN, 1), jnp.float32),
        ],
    )(x, w, deg2)


def _amid_body(acc_ref, dis_ref, b_ref, w_ref, y_ref):
    dis = dis_ref[...]
    ysum = jnp.zeros((BN, DH), jnp.float32)
    for ch in range(NCH):
        h = jnp.maximum(acc_ref[ch] * dis + b_ref[0, ch * CW:(ch + 1) * CW], 0.0)
        ysum += jnp.dot(h, w_ref[ch * CW:(ch + 1) * CW, :],
                        preferred_element_type=jnp.float32)
    y = ysum * dis
    for ch in range(NCH):
        y_ref[ch, :, :] = y[:, ch * CW:(ch + 1) * CW]


def _amid_call(acc, dis, b, w):
    return pl.pallas_call(
        _amid_body,
        grid=(NN // BN,),
        in_specs=[
            pl.BlockSpec((NCH, BN, CW), lambda i: (0, i, 0)),
            pl.BlockSpec((BN, 1), lambda i: (i, 0)),
            pl.BlockSpec((1, DH), lambda i: (0, 0)),
            pl.BlockSpec((DH, DH), lambda i: (0, 0)),
        ],
        out_specs=pl.BlockSpec((NCH, BN, CW), lambda i: (0, i, 0)),
        out_shape=jax.ShapeDtypeStruct((NCH, NNP, CW), jnp.float32),
    )(acc, dis, b, w)


def _head_body(acc_ref, dis_ref, b3_ref, w1_ref, l1b_ref, w2_ref, l2b_ref, o_ref):
    dis = dis_ref[...]
    z = jnp.zeros((BN, DH), jnp.float32)
    for ch in range(NCH):
        h = jnp.maximum(acc_ref[ch] * dis + b3_ref[0, ch * CW:(ch + 1) * CW], 0.0)
        z += jnp.dot(h, w1_ref[ch * CW:(ch + 1) * CW, :],
                     preferred_element_type=jnp.float32)
    z = jnp.maximum(z + l1b_ref[...], 0.0)
    o_ref[...] = jnp.dot(z, w2_ref[...], preferred_element_type=jnp.float32) \
        + l2b_ref[...]


def _head_call(acc, dis, b3, w1, l1b, w2, l2b):
    ncls = w2.shape[1]
    return pl.pallas_call(
        _head_body,
        grid=(NN // BN,),
        in_specs=[
            pl.BlockSpec((NCH, BN, CW), lambda i: (0, i, 0)),
            pl.BlockSpec((BN, 1), lambda i: (i, 0)),
            pl.BlockSpec((1, DH), lambda i: (0, 0)),
            pl.BlockSpec((DH, DH), lambda i: (0, 0)),
            pl.BlockSpec((1, DH), lambda i: (0, 0)),
            pl.BlockSpec((DH, ncls), lambda i: (0, 0)),
            pl.BlockSpec((1, ncls), lambda i: (0, 0)),
        ],
        out_specs=pl.BlockSpec((BN, ncls), lambda i: (i, 0)),
        out_shape=jax.ShapeDtypeStruct((NN, ncls), jnp.float32),
    )(acc, dis, b3, w1, l1b, w2, l2b)


def kernel(dataX, dataY, W1, b1, W2, b2, W3, b3, lin1_W, lin1_b, lin2_W, lin2_b):
    src = dataY[0].astype(jnp.int32)
    dst = dataY[1].astype(jnp.int32)
    pad_rows = jnp.arange(NPAD, dtype=jnp.int32) + NN
    padb = jnp.broadcast_to(pad_rows, (NS, NPAD))
    srcp = jnp.concatenate([src.reshape(NS, EPT), padb], axis=1)
    dstp = jnp.concatenate([dst.reshape(NS, EPT), padb], axis=1)
    srcb = srcp.reshape(NS, NBLK, BLK, EB)
    dstb = dstp.reshape(NS, NBLK, BLK, EB)
    src2c = (srcb[None]
             + (jnp.arange(NCH, dtype=jnp.int32) * NNP)[:, None, None, None, None])
    dst2c = jnp.broadcast_to(dstb[None], (NCH, NS, NBLK, BLK, EB))
    sd2 = jnp.concatenate([src2c, dst2c], axis=3).reshape(-1, EB)
    ones_rows = jnp.ones((EB, CW), jnp.float32)
    zeros_rows = jnp.zeros((EB, CW), jnp.float32)

    deg2 = _deg_call(dst, ones_rows, zeros_rows).reshape(NC, NNP, CW)
    y1, dis = _a1_call(dataX, W1, deg2)
    acc1 = _prop_call(y1.reshape(NCH * NNP, CW), sd2)
    y2 = _amid_call(acc1.reshape(NCH, NNP, CW), dis, b1.reshape(1, DH), W2)
    acc2 = _prop_call(y2.reshape(NCH * NNP, CW), sd2)
    y3 = _amid_call(acc2.reshape(NCH, NNP, CW), dis, b2.reshape(1, DH), W3)
    acc3 = _prop_call(y3.reshape(NCH * NNP, CW), sd2)
    return _head_call(acc3.reshape(NCH, NNP, CW), dis, b3.reshape(1, DH),
                      lin1_W, lin1_b.reshape(1, DH),
                      lin2_W, lin2_b.reshape(1, lin2_W.shape[1]))


# speedup vs baseline: 1.1857x; 1.0003x over previous
import functools

import jax
import jax.numpy as jnp
from jax import lax
from jax.experimental import pallas as pl
from jax.experimental.pallas import tpu as pltpu
from jax.experimental.pallas import tpu_sc as plsc

NN = 10000
NNP = 10240
DH = 512
NCH = 4
CW = 128
EE = 160000
NC = 2
NS = 16
RPT = NNP // NS
EPT = EE // NS
EPD = EE // (NC * NS)
EB = 128
NB = 80
BLK = 20
NBLK = NB // BLK
NPAD = NB * EB - EPT
BN = 2000

_MESH = plsc.VectorSubcoreMesh(
    core_axis_name="c", subcore_axis_name="s", num_cores=NC, num_subcores=NS)


def _deg_body(dst_hbm, ones_hbm, zeros_hbm, out_hbm,
              ones_v, stage_v, idx_v, idxt_v, deg_sh):
    c = lax.axis_index("c")
    s = lax.axis_index("s")
    pltpu.sync_copy(ones_hbm, ones_v)
    pltpu.sync_copy(zeros_hbm, stage_v)
    for j in range(RPT // EB):
        pltpu.sync_copy(stage_v, deg_sh.at[pl.ds(s * RPT + j * EB, EB)])
    plsc.subcore_barrier()
    base = (c * NS + s) * EPD
    nfull = EPD // EB
    tail = EPD - nfull * EB

    def ebody(i, carry):
        pltpu.sync_copy(dst_hbm.at[pl.ds(base + i * EB, EB)], idx_v)
        pltpu.sync_copy(ones_v, deg_sh.at[idx_v], add=True)
        return carry

    lax.fori_loop(0, nfull, ebody, 0)
    pltpu.sync_copy(dst_hbm.at[pl.ds(base + nfull * EB, tail)], idxt_v)
    pltpu.sync_copy(ones_v.at[pl.ds(0, tail)], deg_sh.at[idxt_v], add=True)
    plsc.subcore_barrier()
    for j in range(RPT // EB):
        pltpu.sync_copy(deg_sh.at[pl.ds(s * RPT + j * EB, EB)], stage_v)
        pltpu.sync_copy(stage_v, out_hbm.at[pl.ds(c * NNP + s * RPT + j * EB, EB)])


_deg_call = pl.kernel(
    _deg_body,
    out_type=jax.ShapeDtypeStruct((NC * NNP, CW), jnp.float32),
    mesh=_MESH,
    scratch_types=[
        pltpu.VMEM((EB, CW), jnp.float32),
        pltpu.VMEM((EB, CW), jnp.float32),
        pltpu.VMEM((EB,), jnp.int32),
        pltpu.VMEM((8,), jnp.int32),
        pltpu.VMEM_SHARED((NNP, CW), jnp.float32),
    ],
)


def _prop_body(y_hbm, sd2_hbm, out_hbm,
               rows0, rows1, sdblk, acc_sh, semg0, semg1, sems0, sems1):
    c = lax.axis_index("c")
    s = lax.axis_index("s")
    bufs = (rows0, rows1)
    gsems = (semg0, semg1)
    ssems = (sems0, sems1)
    npc = RPT // EB
    for k in range(NCH // NC):
        chunk = c + NC * k
        ybase = chunk * NNP
        h = pltpu.async_copy(y_hbm.at[pl.ds(ybase + s * RPT, EB)], rows0, semg0)
        for j in range(1, npc + 1):
            if j < npc:
                h2 = pltpu.async_copy(
                    y_hbm.at[pl.ds(ybase + s * RPT + j * EB, EB)],
                    bufs[j % 2], gsems[j % 2])
            h.wait()
            pltpu.sync_copy(bufs[(j - 1) % 2],
                            acc_sh.at[pl.ds(s * RPT + (j - 1) * EB, EB)])
            if j < npc:
                h = h2
        plsc.subcore_barrier()
        sdbase = ((chunk * NS + s) * NBLK) * 2 * BLK

        def blk_body(b, carry):
            pltpu.sync_copy(sd2_hbm.at[pl.ds(sdbase + b * 2 * BLK, 2 * BLK)],
                            sdblk)
            hs = [None, None]
            g = pltpu.async_copy(y_hbm.at[sdblk.at[0]], bufs[0], gsems[0])
            for j in range(BLK):
                p = j % 2
                q = (j + 1) % 2
                if j + 1 < BLK:
                    if hs[q] is not None:
                        hs[q].wait()
                    g2 = pltpu.async_copy(y_hbm.at[sdblk.at[j + 1]],
                                          bufs[q], gsems[q])
                g.wait()
                hs[p] = pltpu.async_copy(bufs[p], acc_sh.at[sdblk.at[BLK + j]],
                                         ssems[p], add=True)
                if j + 1 < BLK:
                    g = g2
            hs[0].wait()
            hs[1].wait()
            return carry

        lax.fori_loop(0, NBLK, blk_body, 0)
        plsc.subcore_barrier()
        h = pltpu.async_copy(acc_sh.at[pl.ds(s * RPT, EB)], rows0, semg0)
        for j in range(1, npc + 1):
            if j < npc:
                h2 = pltpu.async_copy(acc_sh.at[pl.ds(s * RPT + j * EB, EB)],
                                      bufs[j % 2], gsems[j % 2])
            h.wait()
            pltpu.sync_copy(bufs[(j - 1) % 2],
                            out_hbm.at[pl.ds(ybase + s * RPT + (j - 1) * EB, EB)])
            if j < npc:
                h = h2
        if k + 1 < NCH // NC:
            plsc.subcore_barrier()


_prop_call = pl.kernel(
    _prop_body,
    out_type=jax.ShapeDtypeStruct((NCH * NNP, CW), jnp.float32),
    mesh=_MESH,
    scratch_types=[
        pltpu.VMEM((EB, CW), jnp.float32),
        pltpu.VMEM((EB, CW), jnp.float32),
        pltpu.VMEM((2 * BLK, EB), jnp.int32),
        pltpu.VMEM_SHARED((NNP, CW), jnp.float32),
        pltpu.SemaphoreType.DMA,
        pltpu.SemaphoreType.DMA,
        pltpu.SemaphoreType.DMA,
        pltpu.SemaphoreType.DMA,
    ],
)


def _a1_body(x_ref, w_ref, deg_ref, y_ref, dis_ref):
    degsum = deg_ref[0, :, 0:1] + deg_ref[1, :, 0:1] + 1.0
    dis = lax.rsqrt(degsum)
    dis_ref[...] = dis
    xb = x_ref[...].astype(jnp.bfloat16)
    wb = w_ref[...].astype(jnp.bfloat16)
    y = jnp.dot(xb, wb, preferred_element_type=jnp.float32) * dis
    for ch in range(NCH):
        y_ref[ch, :, :] = y[:, ch * CW:(ch + 1) * CW]


def _a1_call(x, w, deg2):
    return pl.pallas_call(
        _a1_body,
        grid=(NN // BN,),
        in_specs=[
            pl.BlockSpec((BN, x.shape[1]), lambda i: (i, 0)),
            pl.BlockSpec(w.shape, lambda i: (0, 0)),
            pl.BlockSpec((NC, BN, CW), lambda i: (0, i, 0)),
        ],
        out_specs=[
            pl.BlockSpec((NCH, BN, CW), lambda i: (0, i, 0)),
            pl.BlockSpec((BN, 1), lambda i: (i, 0)),
        ],
        out_shape=[
            jax.ShapeDtypeStruct((NCH, NNP, CW), jnp.float32),
            jax.ShapeDtypeStruct((NN, 1), jnp.float32),
        ],
    )(x, w, deg2)


def _amid_body(acc_ref, dis_ref, b_ref, w_ref, y_ref):
    dis = dis_ref[...]
    ysum = jnp.zeros((BN, DH), jnp.float32)
    wb = w_ref[...].astype(jnp.bfloat16)
    for ch in range(NCH):
        h = jnp.maximum(acc_ref[ch] * dis + b_ref[0, ch * CW:(ch + 1) * CW], 0.0)
        ysum += jnp.dot(h.astype(jnp.bfloat16), wb[ch * CW:(ch + 1) * CW, :],
                        preferred_element_type=jnp.float32)
    y = ysum * dis
    for ch in range(NCH):
        y_ref[ch, :, :] = y[:, ch * CW:(ch + 1) * CW]


def _amid_call(acc, dis, b, w):
    return pl.pallas_call(
        _amid_body,
        grid=(NN // BN,),
        in_specs=[
            pl.BlockSpec((NCH, BN, CW), lambda i: (0, i, 0)),
            pl.BlockSpec((BN, 1), lambda i: (i, 0)),
            pl.BlockSpec((1, DH), lambda i: (0, 0)),
            pl.BlockSpec((DH, DH), lambda i: (0, 0)),
        ],
        out_specs=pl.BlockSpec((NCH, BN, CW), lambda i: (0, i, 0)),
        out_shape=jax.ShapeDtypeStruct((NCH, NNP, CW), jnp.float32),
    )(acc, dis, b, w)


def _head_body(acc_ref, dis_ref, b3_ref, w1_ref, l1b_ref, w2_ref, l2b_ref, o_ref):
    dis = dis_ref[...]
    z = jnp.zeros((BN, DH), jnp.float32)
    w1b = w1_ref[...].astype(jnp.bfloat16)
    for ch in range(NCH):
        h = jnp.maximum(acc_ref[ch] * dis + b3_ref[0, ch * CW:(ch + 1) * CW], 0.0)
        z += jnp.dot(h.astype(jnp.bfloat16), w1b[ch * CW:(ch + 1) * CW, :],
                     preferred_element_type=jnp.float32)
    z = jnp.maximum(z + l1b_ref[...], 0.0)
    o_ref[...] = jnp.dot(z.astype(jnp.bfloat16),
                         w2_ref[...].astype(jnp.bfloat16),
                         preferred_element_type=jnp.float32) + l2b_ref[...]


def _head_call(acc, dis, b3, w1, l1b, w2, l2b):
    ncls = w2.shape[1]
    return pl.pallas_call(
        _head_body,
        grid=(NN // BN,),
        in_specs=[
            pl.BlockSpec((NCH, BN, CW), lambda i: (0, i, 0)),
            pl.BlockSpec((BN, 1), lambda i: (i, 0)),
            pl.BlockSpec((1, DH), lambda i: (0, 0)),
            pl.BlockSpec((DH, DH), lambda i: (0, 0)),
            pl.BlockSpec((1, DH), lambda i: (0, 0)),
            pl.BlockSpec((DH, ncls), lambda i: (0, 0)),
            pl.BlockSpec((1, ncls), lambda i: (0, 0)),
        ],
        out_specs=pl.BlockSpec((BN, ncls), lambda i: (i, 0)),
        out_shape=jax.ShapeDtypeStruct((NN, ncls), jnp.float32),
    )(acc, dis, b3, w1, l1b, w2, l2b)


def kernel(dataX, dataY, W1, b1, W2, b2, W3, b3, lin1_W, lin1_b, lin2_W, lin2_b):
    src = dataY[0].astype(jnp.int32)
    dst = dataY[1].astype(jnp.int32)
    pad_rows = jnp.arange(NPAD, dtype=jnp.int32) + NN
    padb = jnp.broadcast_to(pad_rows, (NS, NPAD))
    srcp = jnp.concatenate([src.reshape(NS, EPT), padb], axis=1)
    dstp = jnp.concatenate([dst.reshape(NS, EPT), padb], axis=1)
    srcb = srcp.reshape(NS, NBLK, BLK, EB)
    dstb = dstp.reshape(NS, NBLK, BLK, EB)
    src2c = (srcb[None]
             + (jnp.arange(NCH, dtype=jnp.int32) * NNP)[:, None, None, None, None])
    dst2c = jnp.broadcast_to(dstb[None], (NCH, NS, NBLK, BLK, EB))
    sd2 = jnp.concatenate([src2c, dst2c], axis=3).reshape(-1, EB)
    ones_rows = jnp.ones((EB, CW), jnp.float32)
    zeros_rows = jnp.zeros((EB, CW), jnp.float32)

    deg2 = _deg_call(dst, ones_rows, zeros_rows).reshape(NC, NNP, CW)
    y1, dis = _a1_call(dataX, W1, deg2)
    acc1 = _prop_call(y1.reshape(NCH * NNP, CW), sd2)
    y2 = _amid_call(acc1.reshape(NCH, NNP, CW), dis, b1.reshape(1, DH), W2)
    acc2 = _prop_call(y2.reshape(NCH * NNP, CW), sd2)
    y3 = _amid_call(acc2.reshape(NCH, NNP, CW), dis, b2.reshape(1, DH), W3)
    acc3 = _prop_call(y3.reshape(NCH * NNP, CW), sd2)
    return _head_call(acc3.reshape(NCH, NNP, CW), dis, b3.reshape(1, DH),
                      lin1_W, lin1_b.reshape(1, DH),
                      lin2_W, lin2_b.reshape(1, lin2_W.shape[1]))


# speedup vs baseline: 1.2080x; 1.0188x over previous
import functools

import jax
import jax.numpy as jnp
from jax import lax
from jax.experimental import pallas as pl
from jax.experimental.pallas import tpu as pltpu
from jax.experimental.pallas import tpu_sc as plsc

NN = 10000
NNP = 10240
DH = 512
NCH = 4
CW = 128
EE = 160000
NC = 2
NS = 16
RPT = NNP // NS
EPT = EE // NS
EPD = EE // (NC * NS)
EB = 128
NB = 80
BLK = 20
NBLK = NB // BLK
NPAD = NB * EB - EPT
BN = 2000

_MESH = plsc.VectorSubcoreMesh(
    core_axis_name="c", subcore_axis_name="s", num_cores=NC, num_subcores=NS)


def _deg_body(dst_hbm, ones_hbm, zeros_hbm, out_hbm,
              ones_v, stage_v, idx_v, idx1_v, idxt_v, deg_sh, ds0, ds1):
    c = lax.axis_index("c")
    s = lax.axis_index("s")
    pltpu.sync_copy(ones_hbm, ones_v)
    pltpu.sync_copy(zeros_hbm, stage_v)
    for j in range(RPT // EB):
        pltpu.sync_copy(stage_v, deg_sh.at[pl.ds(s * RPT + j * EB, EB)])
    plsc.subcore_barrier()
    base = (c * NS + s) * EPD
    nfull = EPD // EB
    tail = EPD - nfull * EB
    pltpu.sync_copy(dst_hbm.at[pl.ds(base, EB)], idx_v)

    def ebody(g, carry):
        h0 = pltpu.async_copy(ones_v, deg_sh.at[idx_v], ds0, add=True)
        pltpu.sync_copy(dst_hbm.at[pl.ds(base + (2 * g + 1) * EB, EB)], idx1_v)
        h0.wait()
        h1 = pltpu.async_copy(ones_v, deg_sh.at[idx1_v], ds1, add=True)
        pltpu.sync_copy(dst_hbm.at[pl.ds(base + (2 * g + 2) * EB, EB)], idx_v)
        h1.wait()
        return carry

    lax.fori_loop(0, (nfull - 1) // 2, ebody, 0)
    h = pltpu.async_copy(ones_v, deg_sh.at[idx_v], ds0, add=True)
    pltpu.sync_copy(dst_hbm.at[pl.ds(base + nfull * EB, tail)], idxt_v)
    h.wait()
    pltpu.sync_copy(ones_v.at[pl.ds(0, tail)], deg_sh.at[idxt_v], add=True)
    plsc.subcore_barrier()
    for j in range(RPT // EB):
        pltpu.sync_copy(deg_sh.at[pl.ds(s * RPT + j * EB, EB)], stage_v)
        pltpu.sync_copy(stage_v, out_hbm.at[pl.ds(c * NNP + s * RPT + j * EB, EB)])


_deg_call = pl.kernel(
    _deg_body,
    out_type=jax.ShapeDtypeStruct((NC * NNP, CW), jnp.float32),
    mesh=_MESH,
    scratch_types=[
        pltpu.VMEM((EB, CW), jnp.float32),
        pltpu.VMEM((EB, CW), jnp.float32),
        pltpu.VMEM((EB,), jnp.int32),
        pltpu.VMEM((EB,), jnp.int32),
        pltpu.VMEM((8,), jnp.int32),
        pltpu.VMEM_SHARED((NNP, CW), jnp.float32),
        pltpu.SemaphoreType.DMA,
        pltpu.SemaphoreType.DMA,
    ],
)


def _prop_body(y_hbm, sd2_hbm, out_hbm,
               rows0, rows1, sdblk, acc_sh, semg0, semg1, sems0, sems1):
    c = lax.axis_index("c")
    s = lax.axis_index("s")
    bufs = (rows0, rows1)
    gsems = (semg0, semg1)
    ssems = (sems0, sems1)
    npc = RPT // EB
    for k in range(NCH // NC):
        chunk = c + NC * k
        ybase = chunk * NNP
        h = pltpu.async_copy(y_hbm.at[pl.ds(ybase + s * RPT, EB)], rows0, semg0)
        for j in range(1, npc + 1):
            if j < npc:
                h2 = pltpu.async_copy(
                    y_hbm.at[pl.ds(ybase + s * RPT + j * EB, EB)],
                    bufs[j % 2], gsems[j % 2])
            h.wait()
            pltpu.sync_copy(bufs[(j - 1) % 2],
                            acc_sh.at[pl.ds(s * RPT + (j - 1) * EB, EB)])
            if j < npc:
                h = h2
        plsc.subcore_barrier()
        sdbase = ((chunk * NS + s) * NBLK) * 2 * BLK

        def blk_body(b, carry):
            pltpu.sync_copy(sd2_hbm.at[pl.ds(sdbase + b * 2 * BLK, 2 * BLK)],
                            sdblk)
            hs = [None, None]
            g = pltpu.async_copy(y_hbm.at[sdblk.at[0]], bufs[0], gsems[0])
            for j in range(BLK):
                p = j % 2
                q = (j + 1) % 2
                if j + 1 < BLK:
                    if hs[q] is not None:
                        hs[q].wait()
                    g2 = pltpu.async_copy(y_hbm.at[sdblk.at[j + 1]],
                                          bufs[q], gsems[q])
                g.wait()
                hs[p] = pltpu.async_copy(bufs[p], acc_sh.at[sdblk.at[BLK + j]],
                                         ssems[p], add=True)
                if j + 1 < BLK:
                    g = g2
            hs[0].wait()
            hs[1].wait()
            return carry

        lax.fori_loop(0, NBLK, blk_body, 0)
        plsc.subcore_barrier()
        h = pltpu.async_copy(acc_sh.at[pl.ds(s * RPT, EB)], rows0, semg0)
        for j in range(1, npc + 1):
            if j < npc:
                h2 = pltpu.async_copy(acc_sh.at[pl.ds(s * RPT + j * EB, EB)],
                                      bufs[j % 2], gsems[j % 2])
            h.wait()
            pltpu.sync_copy(bufs[(j - 1) % 2],
                            out_hbm.at[pl.ds(ybase + s * RPT + (j - 1) * EB, EB)])
            if j < npc:
                h = h2
        if k + 1 < NCH // NC:
            plsc.subcore_barrier()


_prop_call = pl.kernel(
    _prop_body,
    out_type=jax.ShapeDtypeStruct((NCH * NNP, CW), jnp.float32),
    mesh=_MESH,
    scratch_types=[
        pltpu.VMEM((EB, CW), jnp.float32),
        pltpu.VMEM((EB, CW), jnp.float32),
        pltpu.VMEM((2 * BLK, EB), jnp.int32),
        pltpu.VMEM_SHARED((NNP, CW), jnp.float32),
        pltpu.SemaphoreType.DMA,
        pltpu.SemaphoreType.DMA,
        pltpu.SemaphoreType.DMA,
        pltpu.SemaphoreType.DMA,
    ],
)


def _a1_body(x_ref, w_ref, deg_ref, y_ref, dis_ref):
    degsum = deg_ref[0, :, 0:1] + deg_ref[1, :, 0:1] + 1.0
    dis = lax.rsqrt(degsum)
    dis_ref[...] = dis
    y = jnp.dot(x_ref[...], w_ref[...], preferred_element_type=jnp.float32) * dis
    for ch in range(NCH):
        y_ref[ch, :, :] = y[:, ch * CW:(ch + 1) * CW]


def _a1_call(x, w, deg2):
    return pl.pallas_call(
        _a1_body,
        grid=(NN // BN,),
        in_specs=[
            pl.BlockSpec((BN, x.shape[1]), lambda i: (i, 0)),
            pl.BlockSpec(w.shape, lambda i: (0, 0)),
            pl.BlockSpec((NC, BN, CW), lambda i: (0, i, 0)),
        ],
        out_specs=[
            pl.BlockSpec((NCH, BN, CW), lambda i: (0, i, 0)),
            pl.BlockSpec((BN, 1), lambda i: (i, 0)),
        ],
        out_shape=[
            jax.ShapeDtypeStruct((NCH, NNP, CW), jnp.float32),
            jax.ShapeDtypeStruct((NN, 1), jnp.float32),
        ],
    )(x, w, deg2)


def _amid_body(acc_ref, dis_ref, b_ref, w_ref, y_ref):
    dis = dis_ref[...]
    ysum = jnp.zeros((BN, DH), jnp.float32)
    for ch in range(NCH):
        h = jnp.maximum(acc_ref[ch] * dis + b_ref[0, ch * CW:(ch + 1) * CW], 0.0)
        ysum += jnp.dot(h, w_ref[ch * CW:(ch + 1) * CW, :],
                        preferred_element_type=jnp.float32)
    y = ysum * dis
    for ch in range(NCH):
        y_ref[ch, :, :] = y[:, ch * CW:(ch + 1) * CW]


def _amid_call(acc, dis, b, w):
    return pl.pallas_call(
        _amid_body,
        grid=(NN // BN,),
        in_specs=[
            pl.BlockSpec((NCH, BN, CW), lambda i: (0, i, 0)),
            pl.BlockSpec((BN, 1), lambda i: (i, 0)),
            pl.BlockSpec((1, DH), lambda i: (0, 0)),
            pl.BlockSpec((DH, DH), lambda i: (0, 0)),
        ],
        out_specs=pl.BlockSpec((NCH, BN, CW), lambda i: (0, i, 0)),
        out_shape=jax.ShapeDtypeStruct((NCH, NNP, CW), jnp.float32),
    )(acc, dis, b, w)


def _head_body(acc_ref, dis_ref, b3_ref, w1_ref, l1b_ref, w2_ref, l2b_ref, o_ref):
    dis = dis_ref[...]
    z = jnp.zeros((BN, DH), jnp.float32)
    for ch in range(NCH):
        h = jnp.maximum(acc_ref[ch] * dis + b3_ref[0, ch * CW:(ch + 1) * CW], 0.0)
        z += jnp.dot(h, w1_ref[ch * CW:(ch + 1) * CW, :],
                     preferred_element_type=jnp.float32)
    z = jnp.maximum(z + l1b_ref[...], 0.0)
    o_ref[...] = jnp.dot(z, w2_ref[...], preferred_element_type=jnp.float32) \
        + l2b_ref[...]


def _head_call(acc, dis, b3, w1, l1b, w2, l2b):
    ncls = w2.shape[1]
    return pl.pallas_call(
        _head_body,
        grid=(NN // BN,),
        in_specs=[
            pl.BlockSpec((NCH, BN, CW), lambda i: (0, i, 0)),
            pl.BlockSpec((BN, 1), lambda i: (i, 0)),
            pl.BlockSpec((1, DH), lambda i: (0, 0)),
            pl.BlockSpec((DH, DH), lambda i: (0, 0)),
            pl.BlockSpec((1, DH), lambda i: (0, 0)),
            pl.BlockSpec((DH, ncls), lambda i: (0, 0)),
            pl.BlockSpec((1, ncls), lambda i: (0, 0)),
        ],
        out_specs=pl.BlockSpec((BN, ncls), lambda i: (i, 0)),
        out_shape=jax.ShapeDtypeStruct((NN, ncls), jnp.float32),
    )(acc, dis, b3, w1, l1b, w2, l2b)


def kernel(dataX, dataY, W1, b1, W2, b2, W3, b3, lin1_W, lin1_b, lin2_W, lin2_b):
    src = dataY[0].astype(jnp.int32)
    dst = dataY[1].astype(jnp.int32)
    pad_rows = jnp.arange(NPAD, dtype=jnp.int32) + NN
    padb = jnp.broadcast_to(pad_rows, (NS, NPAD))
    srcp = jnp.concatenate([src.reshape(NS, EPT), padb], axis=1)
    dstp = jnp.concatenate([dst.reshape(NS, EPT), padb], axis=1)
    srcb = srcp.reshape(NS, NBLK, BLK, EB)
    dstb = dstp.reshape(NS, NBLK, BLK, EB)
    src2c = (srcb[None]
             + (jnp.arange(NCH, dtype=jnp.int32) * NNP)[:, None, None, None, None])
    dst2c = jnp.broadcast_to(dstb[None], (NCH, NS, NBLK, BLK, EB))
    sd2 = jnp.concatenate([src2c, dst2c], axis=3).reshape(-1, EB)
    ones_rows = jnp.ones((EB, CW), jnp.float32)
    zeros_rows = jnp.zeros((EB, CW), jnp.float32)

    deg2 = _deg_call(dst, ones_rows, zeros_rows).reshape(NC, NNP, CW)
    y1, dis = _a1_call(dataX, W1, deg2)
    acc1 = _prop_call(y1.reshape(NCH * NNP, CW), sd2)
    y2 = _amid_call(acc1.reshape(NCH, NNP, CW), dis, b1.reshape(1, DH), W2)
    acc2 = _prop_call(y2.reshape(NCH * NNP, CW), sd2)
    y3 = _amid_call(acc2.reshape(NCH, NNP, CW), dis, b2.reshape(1, DH), W3)
    acc3 = _prop_call(y3.reshape(NCH * NNP, CW), sd2)
    return _head_call(acc3.reshape(NCH, NNP, CW), dis, b3.reshape(1, DH),
                      lin1_W, lin1_b.reshape(1, DH),
                      lin2_W, lin2_b.reshape(1, lin2_W.shape[1]))


# speedup vs baseline: 1.2101x; 1.0018x over previous
import jax
import jax.numpy as jnp
from jax import lax
from jax.experimental import pallas as pl
from jax.experimental.pallas import tpu as pltpu
from jax.experimental.pallas import tpu_sc as plsc

NN = 10000
NNP = 10240
DH = 512
NCH = 4
CW = 128
EE = 160000
NC = 2
NS = 16
RPT = NNP // NS
EPT = EE // NS
EPD = EE // (NC * NS)
EB = 128
NB = 80
BLK = 20
NBLK = NB // BLK
NPAD = NB * EB - EPT
BN = 2000

_MESH = plsc.VectorSubcoreMesh(
    core_axis_name="c", subcore_axis_name="s", num_cores=NC, num_subcores=NS)


def _deg_body(dst_hbm, ones_hbm, zeros_hbm, out_hbm,
              ones_v, stage_v, idx_v, idx1_v, idxt_v, deg_sh, ds0, ds1):
    c = lax.axis_index("c")
    s = lax.axis_index("s")
    pltpu.sync_copy(ones_hbm, ones_v)
    pltpu.sync_copy(zeros_hbm, stage_v)
    for j in range(RPT // EB):
        pltpu.sync_copy(stage_v, deg_sh.at[pl.ds(s * RPT + j * EB, EB)])
    plsc.subcore_barrier()
    base = (c * NS + s) * EPD
    nfull = EPD // EB
    tail = EPD - nfull * EB
    pltpu.sync_copy(dst_hbm.at[pl.ds(base, EB)], idx_v)

    def ebody(g, carry):
        h0 = pltpu.async_copy(ones_v, deg_sh.at[idx_v], ds0, add=True)
        pltpu.sync_copy(dst_hbm.at[pl.ds(base + (2 * g + 1) * EB, EB)], idx1_v)
        h0.wait()
        h1 = pltpu.async_copy(ones_v, deg_sh.at[idx1_v], ds1, add=True)
        pltpu.sync_copy(dst_hbm.at[pl.ds(base + (2 * g + 2) * EB, EB)], idx_v)
        h1.wait()
        return carry

    lax.fori_loop(0, (nfull - 1) // 2, ebody, 0)
    h = pltpu.async_copy(ones_v, deg_sh.at[idx_v], ds0, add=True)
    pltpu.sync_copy(dst_hbm.at[pl.ds(base + nfull * EB, tail)], idxt_v)
    h.wait()
    pltpu.sync_copy(ones_v.at[pl.ds(0, tail)], deg_sh.at[idxt_v], add=True)
    plsc.subcore_barrier()
    for j in range(RPT // EB):
        pltpu.sync_copy(deg_sh.at[pl.ds(s * RPT + j * EB, EB)], stage_v)
        pltpu.sync_copy(stage_v, out_hbm.at[pl.ds(c * NNP + s * RPT + j * EB, EB)])


_deg_call = pl.kernel(
    _deg_body,
    out_type=jax.ShapeDtypeStruct((NC * NNP, CW), jnp.float32),
    mesh=_MESH,
    scratch_types=[
        pltpu.VMEM((EB, CW), jnp.float32),
        pltpu.VMEM((EB, CW), jnp.float32),
        pltpu.VMEM((EB,), jnp.int32),
        pltpu.VMEM((EB,), jnp.int32),
        pltpu.VMEM((8,), jnp.int32),
        pltpu.VMEM_SHARED((NNP, CW), jnp.float32),
        pltpu.SemaphoreType.DMA,
        pltpu.SemaphoreType.DMA,
    ],
)


def _prop_body(y_hbm, sd2_hbm, out_hbm,
               rows0, rows1, sdblk, acc_sh, semg0, semg1, sems0, sems1):
    c = lax.axis_index("c")
    s = lax.axis_index("s")
    bufs = (rows0, rows1)
    gsems = (semg0, semg1)
    ssems = (sems0, sems1)
    npc = RPT // EB
    for k in range(NCH // NC):
        chunk = c + NC * k
        ybase = chunk * NNP
        h = pltpu.async_copy(y_hbm.at[pl.ds(ybase + s * RPT, EB)], rows0, semg0)
        for j in range(1, npc + 1):
            if j < npc:
                h2 = pltpu.async_copy(
                    y_hbm.at[pl.ds(ybase + s * RPT + j * EB, EB)],
                    bufs[j % 2], gsems[j % 2])
            h.wait()
            pltpu.sync_copy(bufs[(j - 1) % 2],
                            acc_sh.at[pl.ds(s * RPT + (j - 1) * EB, EB)])
            if j < npc:
                h = h2
        plsc.subcore_barrier()
        sdbase = ((chunk * NS + s) * NBLK) * 2 * BLK

        def blk_body(b, carry):
            pltpu.sync_copy(sd2_hbm.at[pl.ds(sdbase + b * 2 * BLK, 2 * BLK)],
                            sdblk)
            hs = [None, None]
            g = pltpu.async_copy(y_hbm.at[sdblk.at[0]], bufs[0], gsems[0])
            for j in range(BLK):
                p = j % 2
                q = (j + 1) % 2
                if j + 1 < BLK:
                    if hs[q] is not None:
                        hs[q].wait()
                    g2 = pltpu.async_copy(y_hbm.at[sdblk.at[j + 1]],
                                          bufs[q], gsems[q])
                g.wait()
                hs[p] = pltpu.async_copy(bufs[p], acc_sh.at[sdblk.at[BLK + j]],
                                         ssems[p], add=True)
                if j + 1 < BLK:
                    g = g2
            hs[0].wait()
            hs[1].wait()
            return carry

        lax.fori_loop(0, NBLK, blk_body, 0)
        plsc.subcore_barrier()
        h = pltpu.async_copy(acc_sh.at[pl.ds(s * RPT, EB)], rows0, semg0)
        for j in range(1, npc + 1):
            if j < npc:
                h2 = pltpu.async_copy(acc_sh.at[pl.ds(s * RPT + j * EB, EB)],
                                      bufs[j % 2], gsems[j % 2])
            h.wait()
            pltpu.sync_copy(bufs[(j - 1) % 2],
                            out_hbm.at[pl.ds(ybase + s * RPT + (j - 1) * EB, EB)])
            if j < npc:
                h = h2
        if k + 1 < NCH // NC:
            plsc.subcore_barrier()


_prop_call = pl.kernel(
    _prop_body,
    out_type=jax.ShapeDtypeStruct((NCH * NNP, CW), jnp.float32),
    mesh=_MESH,
    scratch_types=[
        pltpu.VMEM((EB, CW), jnp.float32),
        pltpu.VMEM((EB, CW), jnp.float32),
        pltpu.VMEM((2 * BLK, EB), jnp.int32),
        pltpu.VMEM_SHARED((NNP, CW), jnp.float32),
        pltpu.SemaphoreType.DMA,
        pltpu.SemaphoreType.DMA,
        pltpu.SemaphoreType.DMA,
        pltpu.SemaphoreType.DMA,
    ],
)


def _a1_body(x_ref, w_ref, deg_ref, y_ref, dis_ref):
    degsum = deg_ref[0, :, 0:1] + deg_ref[1, :, 0:1] + 1.0
    dis = lax.rsqrt(degsum)
    dis_ref[...] = dis
    y = jnp.dot(x_ref[...], w_ref[...], preferred_element_type=jnp.float32) * dis
    for ch in range(NCH):
        y_ref[ch, :, :] = y[:, ch * CW:(ch + 1) * CW]


def _a1_call(x, w, deg2):
    return pl.pallas_call(
        _a1_body,
        grid=(NN // BN,),
        in_specs=[
            pl.BlockSpec((BN, x.shape[1]), lambda i: (i, 0)),
            pl.BlockSpec(w.shape, lambda i: (0, 0)),
            pl.BlockSpec((NC, BN, CW), lambda i: (0, i, 0)),
        ],
        out_specs=[
            pl.BlockSpec((NCH, BN, CW), lambda i: (0, i, 0)),
            pl.BlockSpec((BN, 1), lambda i: (i, 0)),
        ],
        out_shape=[
            jax.ShapeDtypeStruct((NCH, NNP, CW), jnp.float32),
            jax.ShapeDtypeStruct((NN, 1), jnp.float32),
        ],
    )(x, w, deg2)


def _amid_body(acc_ref, dis_ref, b_ref, w_ref, y_ref):
    dis = dis_ref[...]
    ysum = jnp.zeros((BN, DH), jnp.float32)
    for ch in range(NCH):
        h = jnp.maximum(acc_ref[ch] * dis + b_ref[0, ch * CW:(ch + 1) * CW], 0.0)
        ysum += jnp.dot(h, w_ref[ch * CW:(ch + 1) * CW, :],
                        preferred_element_type=jnp.float32)
    y = ysum * dis
    for ch in range(NCH):
        y_ref[ch, :, :] = y[:, ch * CW:(ch + 1) * CW]


def _amid_call(acc, dis, b, w):
    return pl.pallas_call(
        _amid_body,
        grid=(NN // BN,),
        in_specs=[
            pl.BlockSpec((NCH, BN, CW), lambda i: (0, i, 0)),
            pl.BlockSpec((BN, 1), lambda i: (i, 0)),
            pl.BlockSpec((1, DH), lambda i: (0, 0)),
            pl.BlockSpec((DH, DH), lambda i: (0, 0)),
        ],
        out_specs=pl.BlockSpec((NCH, BN, CW), lambda i: (0, i, 0)),
        out_shape=jax.ShapeDtypeStruct((NCH, NNP, CW), jnp.float32),
    )(acc, dis, b, w)


def _head_body(acc_ref, dis_ref, b3_ref, w1_ref, l1b_ref, w2_ref, l2b_ref, o_ref):
    dis = dis_ref[...]
    z = jnp.zeros((BN, DH), jnp.float32)
    for ch in range(NCH):
        h = jnp.maximum(acc_ref[ch] * dis + b3_ref[0, ch * CW:(ch + 1) * CW], 0.0)
        z += jnp.dot(h, w1_ref[ch * CW:(ch + 1) * CW, :],
                     preferred_element_type=jnp.float32)
    z = jnp.maximum(z + l1b_ref[...], 0.0)
    o_ref[...] = jnp.dot(z, w2_ref[...], preferred_element_type=jnp.float32) \
        + l2b_ref[...]


def _head_call(acc, dis, b3, w1, l1b, w2, l2b):
    ncls = w2.shape[1]
    return pl.pallas_call(
        _head_body,
        grid=(NN // BN,),
        in_specs=[
            pl.BlockSpec((NCH, BN, CW), lambda i: (0, i, 0)),
            pl.BlockSpec((BN, 1), lambda i: (i, 0)),
            pl.BlockSpec((1, DH), lambda i: (0, 0)),
            pl.BlockSpec((DH, DH), lambda i: (0, 0)),
            pl.BlockSpec((1, DH), lambda i: (0, 0)),
            pl.BlockSpec((DH, ncls), lambda i: (0, 0)),
            pl.BlockSpec((1, ncls), lambda i: (0, 0)),
        ],
        out_specs=pl.BlockSpec((BN, ncls), lambda i: (i, 0)),
        out_shape=jax.ShapeDtypeStruct((NN, ncls), jnp.float32),
    )(acc, dis, b3, w1, l1b, w2, l2b)


def kernel(dataX, dataY, W1, b1, W2, b2, W3, b3, lin1_W, lin1_b, lin2_W, lin2_b):
    src = dataY[0].astype(jnp.int32)
    dst = dataY[1].astype(jnp.int32)
    pad_rows = jnp.arange(NPAD, dtype=jnp.int32) + NN
    padb = jnp.broadcast_to(pad_rows, (NS, NPAD))
    srcp = jnp.concatenate([src.reshape(NS, EPT), padb], axis=1)
    dstp = jnp.concatenate([dst.reshape(NS, EPT), padb], axis=1)
    srcb = srcp.reshape(NS, NBLK, BLK, EB)
    dstb = dstp.reshape(NS, NBLK, BLK, EB)
    src2c = (srcb[None]
             + (jnp.arange(NCH, dtype=jnp.int32) * NNP)[:, None, None, None, None])
    dst2c = jnp.broadcast_to(dstb[None], (NCH, NS, NBLK, BLK, EB))
    sd2 = jnp.concatenate([src2c, dst2c], axis=3).reshape(-1, EB)
    ones_rows = jnp.ones((EB, CW), jnp.float32)
    zeros_rows = jnp.zeros((EB, CW), jnp.float32)

    deg2 = _deg_call(dst, ones_rows, zeros_rows).reshape(NC, NNP, CW)
    y1, dis = _a1_call(dataX, W1, deg2)
    acc1 = _prop_call(y1.reshape(NCH * NNP, CW), sd2)
    y2 = _amid_call(acc1.reshape(NCH, NNP, CW), dis, b1.reshape(1, DH), W2)
    acc2 = _prop_call(y2.reshape(NCH * NNP, CW), sd2)
    y3 = _amid_call(acc2.reshape(NCH, NNP, CW), dis, b2.reshape(1, DH), W3)
    acc3 = _prop_call(y3.reshape(NCH * NNP, CW), sd2)
    return _head_call(acc3.reshape(NCH, NNP, CW), dis, b3.reshape(1, DH),
                      lin1_W, lin1_b.reshape(1, DH),
                      lin2_W, lin2_b.reshape(1, lin2_W.shape[1]))
